# Initial kernel scaffold; baseline (speedup 1.0000x reference)
import jax
import jax.numpy as jnp
from jax.experimental import pallas as pl


def kernel(x, edge_index, edge_attr, W11, b11, W12, b12, W13, b13, g11, be11, g12, be12, g13, be13, g21, be21, g22, be22, g23, be23, Wp, bpb, W21, b21, W22, b22, W23, b23, Wf1, bf1, Wf2, bf2):
    raise NotImplementedError("write your pallas kernel here")


# speedup vs baseline: 18.0783x; 18.0783x over previous
import jax
import jax.numpy as jnp
from jax import lax
from jax.experimental import pallas as pl
from jax.experimental.pallas import tpu as pltpu

G = 20
NPG = 500
N = G * NPG
E = 160000
K = 100

_BETAS = (0.4, 0.3, 0.3, 0.3, 0.2, 0.2, 0.2)
_QS = tuple(range(2, 9))
_QTOT = sum(_QS)
_EPS_BN = 1e-5


def _ctr(a, y, adim, ydim):
    return lax.dot_general(a, y, (((adim,), (ydim,)), ((), ())),
                           precision=lax.Precision.HIGHEST,
                           preferred_element_type=jnp.float32)


def _mmh(a, b):
    return jnp.dot(a, b, precision=lax.Precision.HIGHEST,
                   preferred_element_type=jnp.float32)


def _rd(ref, g):
    return ref[pl.ds(g, 1)][0]


def _wr(ref, g, val):
    ref[pl.ds(g, 1)] = val[None]


def _mega(A_ref, x_ref, b0_ref, logb0_ref, betav_ref, mseg_ref,
          W11_ref, b11_ref, W12_ref, b12_ref, W13_ref, b13_ref,
          g11_ref, be11_ref, g12_ref, be12_ref, g13_ref, be13_ref,
          g21_ref, be21_ref, g22_ref, be22_ref, g23_ref, be23_ref,
          Wp_ref, bpb_ref, W21_ref, b21_ref, W22_ref, b22_ref,
          W23_ref, b23_ref, Wf1_ref, bf1_ref, Wf2_ref, bf2_ref,
          out_ref, reg_ref,
          nrm_ref, agg_ref, hx1_ref, hx2_ref, hx3_ref, bel_ref,
          p1x_ref, padj_ref, agg2_ref, h21_ref, h22_ref, h23_ref,
          x1o_ref, x2o_ref):
    ones_col = jnp.ones((NPG, 1), jnp.float32)

    def nrm_body(g, c):
        a = _rd(A_ref, g)
        deg = _ctr(a, ones_col, 0, 0) + 1.0
        _wr(nrm_ref, g, lax.rsqrt(deg))
        return c
    lax.fori_loop(0, G, nrm_body, 0)

    W11 = W11_ref[...]; b11 = b11_ref[...]

    def gcn_agg(g, xw):
        a = _rd(A_ref, g)
        nc = _rd(nrm_ref, g)
        aggT = _ctr(a, xw * nc, 0, 0)
        return aggT * nc + xw * (nc * nc)

    def l1_body(g, s):
        agg = gcn_agg(g, _mmh(_rd(x_ref, g), W11)) + b11
        _wr(agg_ref, g, agg)
        return s + jnp.sum(agg, axis=0, keepdims=True)
    z30 = jnp.zeros((1, 30), jnp.float32)
    s = lax.fori_loop(0, G, l1_body, z30)

    def var_of(ref, m, n):
        def vb(g, ss):
            d = _rd(ref, g) - m
            return ss + jnp.sum(d * d, axis=0, keepdims=True)
        return lax.fori_loop(0, G, vb, z30) / n

    def bn_coefs(s, ga, be, ref, n):
        m = s / n
        v = var_of(ref, m, n)
        sc = lax.rsqrt(v + _EPS_BN) * ga
        return sc, be - m * sc

    def boundary(s, ga, be, hx_ref, x1o_col, Wn, bn_, more):
        sc, sh = bn_coefs(s, ga, be, agg_ref, N)

        def body(g, s2):
            hx = _rd(agg_ref, g) * sc + sh
            _wr(hx_ref, g, hx)
            x1o_ref[pl.ds(g, 1), pl.ds(x1o_col, 30)] = jnp.max(
                hx, axis=0, keepdims=True)
            if not more:
                return s2
            agg = gcn_agg(g, _mmh(hx, Wn)) + bn_
            _wr(agg_ref, g, agg)
            return s2 + jnp.sum(agg, axis=0, keepdims=True)
        return lax.fori_loop(0, G, body, z30)

    s = boundary(s, g11_ref[...], be11_ref[...], hx1_ref, 0,
                 W12_ref[...], b12_ref[...], True)
    s = boundary(s, g12_ref[...], be12_ref[...], hx2_ref, 30,
                 W13_ref[...], b13_ref[...], True)
    boundary(s, g13_ref[...], be13_ref[...], hx3_ref, 60,
             None, None, False)

    bel_ref[...] = b0_ref[...]
    betav = betav_ref[...]
    mseg = mseg_ref[...]

    def bp_body(g, c):
        a = _rd(A_ref, g)
        lb0 = _rd(logb0_ref, g)
        bel = _rd(bel_ref, g)
        h = _ctr(a, bel, 0, 0)
        e = jnp.exp(betav * h + lb0)
        _wr(bel_ref, g, e / _mmh(e, mseg))
        return c
    for _ in range(5):
        lax.fori_loop(0, G, bp_body, 0)

    Wp = Wp_ref[...]; bpb = bpb_ref[...]
    eyeK = (lax.broadcasted_iota(jnp.int32, (K, K), 0) ==
            lax.broadcasted_iota(jnp.int32, (K, K), 1)).astype(jnp.float32)

    def pool_body(g, carry):
        ml, ent = carry
        a = _rd(A_ref, g)
        S = _mmh(_rd(bel_ref, g), Wp) + bpb
        t = _ctr(a, S, 1, 0)
        ml = ml + jnp.sum(t * S)
        mx = jnp.max(S, axis=-1, keepdims=True)
        es = jnp.exp(S - mx)
        sm = es / jnp.sum(es, axis=-1, keepdims=True)
        ent = ent + jnp.sum(-sm * jnp.log(sm + 1e-15))
        _wr(p1x_ref, g, _ctr(sm, _rd(hx3_ref, g), 0, 0))
        _wr(padj_ref, g, _ctr(sm, _ctr(a, sm, 1, 0), 0, 0))
        return ml, ent
    ml, ent = lax.fori_loop(0, G, pool_body,
                            (jnp.float32(0.0), jnp.float32(0.0)))
    p1_ml = -ml / E
    p1_el = ent / N

    ones_colK = jnp.ones((K, 1), jnp.float32)

    def dgcn_agg(g, z):
        a2 = _rd(padj_ref, g) + eyeK
        nr = lax.rsqrt(_ctr(a2, ones_colK, 1, 0) + 1e-9)
        return nr * _ctr(a2, nr * z, 1, 0)

    def d1_body(g, s2):
        agg = dgcn_agg(g, _mmh(_rd(p1x_ref, g), W21_ref[...])) + b21_ref[...]
        _wr(agg2_ref, g, agg)
        return s2 + jnp.sum(agg, axis=0, keepdims=True)
    s = lax.fori_loop(0, G, d1_body, z30)

    NK = G * K

    def dboundary(s, ga, be, hx_ref, x2o_col, Wn, bn_, more):
        sc, sh = bn_coefs(s, ga, be, agg2_ref, NK)

        def body(g, s2):
            hx = _rd(agg2_ref, g) * sc + sh
            _wr(hx_ref, g, hx)
            x2o_ref[pl.ds(g, 1), pl.ds(x2o_col, 30)] = jnp.max(
                hx, axis=0, keepdims=True)
            if not more:
                return s2
            agg = dgcn_agg(g, _mmh(hx, Wn)) + bn_
            _wr(agg2_ref, g, agg)
            return s2 + jnp.sum(agg, axis=0, keepdims=True)
        return lax.fori_loop(0, G, body, z30)

    s = dboundary(s, g21_ref[...], be21_ref[...], h21_ref, 0,
                  W22_ref[...], b22_ref[...], True)
    s = dboundary(s, g22_ref[...], be22_ref[...], h22_ref, 30,
                  W23_ref[...], b23_ref[...], True)
    dboundary(s, g23_ref[...], be23_ref[...], h23_ref, 60,
              None, None, False)

    conv_out = jnp.concatenate([x1o_ref[...], x2o_ref[...]], axis=-1)
    hid = jnp.maximum(_mmh(conv_out, Wf1_ref[...]) + bf1_ref[...], 0.0)
    out_ref[...] = _mmh(hid, Wf2_ref[...]) + bf2_ref[...]
    reg_ref[...] = jnp.reshape(p1_el + p1_ml, (1, 1))


def kernel(x, edge_index, edge_attr, W11, b11, W12, b12, W13, b13,
           g11, be11, g12, be12, g13, be13, g21, be21, g22, be22, g23, be23,
           Wp, bpb, W21, b21, W22, b22, W23, b23, Wf1, bf1, Wf2, bf2):
    src = edge_index[0].astype(jnp.int32)
    dst = edge_index[1].astype(jnp.int32)
    ew = edge_attr

    gid = src // NPG
    A = jnp.zeros((G, NPG, NPG), jnp.float32).at[gid, src % NPG, dst % NPG].add(ew)

    idx = jnp.arange(N, dtype=jnp.float32)[:, None]
    b0s = []
    for q in _QS:
        freqs = jnp.arange(1, q + 1, dtype=jnp.float32)[None, :]
        z = jnp.sin(idx * freqs * 0.7331)
        z = z - jnp.max(z, axis=-1, keepdims=True)
        ez = jnp.exp(z)
        b0s.append(ez / jnp.sum(ez, axis=-1, keepdims=True))
    b0 = jnp.concatenate(b0s, axis=-1).reshape(G, NPG, _QTOT)
    logb0 = jnp.log(b0 + 1e-9)
    betav = jnp.concatenate([jnp.full((q,), b, jnp.float32)
                             for q, b in zip(_QS, _BETAS)])[None]
    run_id = jnp.concatenate([jnp.full((q,), i, jnp.int32)
                              for i, q in enumerate(_QS)])
    mseg = (run_id[:, None] == run_id[None, :]).astype(jnp.float32)

    f32 = jnp.float32
    scratch = [
        pltpu.VMEM((G, NPG, 1), f32),
        pltpu.VMEM((G, NPG, 30), f32),
        pltpu.VMEM((G, NPG, 30), f32),
        pltpu.VMEM((G, NPG, 30), f32),
        pltpu.VMEM((G, NPG, 30), f32),
        pltpu.VMEM((G, NPG, _QTOT), f32),
        pltpu.VMEM((G, K, 30), f32),
        pltpu.VMEM((G, K, K), f32),
        pltpu.VMEM((G, K, 30), f32),
        pltpu.VMEM((G, K, 30), f32),
        pltpu.VMEM((G, K, 30), f32),
        pltpu.VMEM((G, K, 30), f32),
        pltpu.VMEM((G, 90), f32),
        pltpu.VMEM((G, 90), f32),
    ]

    out, reg = pl.pallas_call(
        _mega,
        out_shape=(jax.ShapeDtypeStruct((G, 6), jnp.float32),
                   jax.ShapeDtypeStruct((1, 1), jnp.float32)),
        scratch_shapes=scratch,
        compiler_params=pltpu.CompilerParams(
            vmem_limit_bytes=100 * 1024 * 1024),
    )(A, x.reshape(G, NPG, 3), b0, logb0, betav, mseg,
      W11, b11, W12, b12, W13, b13,
      g11, be11, g12, be12, g13, be13,
      g21, be21, g22, be22, g23, be23,
      Wp, bpb, W21, b21, W22, b22, W23, b23, Wf1, bf1, Wf2, bf2)
    return (out, jnp.reshape(reg, ()))


# speedup vs baseline: 19.1506x; 1.0593x over previous
import jax
import jax.numpy as jnp
from jax import lax
from jax.experimental import pallas as pl
from jax.experimental.pallas import tpu as pltpu

G = 20
NPG = 500
N = G * NPG
E = 160000
K = 100

_BETAS = (0.4, 0.3, 0.3, 0.3, 0.2, 0.2, 0.2)
_QS = tuple(range(2, 9))
_QTOT = sum(_QS)
_EPS_BN = 1e-5


def _ctr(a, y, adim, ydim, prec=lax.Precision.HIGHEST):
    return lax.dot_general(a, y, (((adim,), (ydim,)), ((), ())),
                           precision=prec,
                           preferred_element_type=jnp.float32)


def _mmh(a, b, prec=lax.Precision.HIGHEST):
    return jnp.dot(a, b, precision=prec,
                   preferred_element_type=jnp.float32)


_DEF = lax.Precision.DEFAULT


def _rd(ref, g):
    return ref[pl.ds(g, 1)][0]


def _wr(ref, g, val):
    ref[pl.ds(g, 1)] = val[None]


def _mega(A_ref, x_ref, b0_ref, logb0_ref, betav_ref, mseg_ref,
          W11_ref, b11_ref, W12_ref, b12_ref, W13_ref, b13_ref,
          g11_ref, be11_ref, g12_ref, be12_ref, g13_ref, be13_ref,
          g21_ref, be21_ref, g22_ref, be22_ref, g23_ref, be23_ref,
          Wp_ref, bpb_ref, W21_ref, b21_ref, W22_ref, b22_ref,
          W23_ref, b23_ref, Wf1_ref, bf1_ref, Wf2_ref, bf2_ref,
          out_ref, reg_ref,
          nrm_ref, agg_ref, hx1_ref, hx2_ref, hx3_ref, bel_ref,
          p1x_ref, padj_ref, agg2_ref, h21_ref, h22_ref, h23_ref,
          x1o_ref, x2o_ref):
    ones_col = jnp.ones((NPG, 1), jnp.float32)

    def nrm_body(g, c):
        a = _rd(A_ref, g)
        deg = _ctr(a, ones_col, 0, 0) + 1.0
        _wr(nrm_ref, g, lax.rsqrt(deg))
        return c
    lax.fori_loop(0, G, nrm_body, 0)

    W11 = W11_ref[...]; b11 = b11_ref[...]

    def gcn_agg(g, xw):
        a = _rd(A_ref, g)
        nc = _rd(nrm_ref, g)
        aggT = _ctr(a, xw * nc, 0, 0)
        return aggT * nc + xw * (nc * nc)

    def l1_body(g, s):
        agg = gcn_agg(g, _mmh(_rd(x_ref, g), W11)) + b11
        _wr(agg_ref, g, agg)
        return s + jnp.sum(agg, axis=0, keepdims=True)
    z30 = jnp.zeros((1, 30), jnp.float32)
    s = lax.fori_loop(0, G, l1_body, z30)

    def var_of(ref, m, n):
        def vb(g, ss):
            d = _rd(ref, g) - m
            return ss + jnp.sum(d * d, axis=0, keepdims=True)
        return lax.fori_loop(0, G, vb, z30) / n

    def bn_coefs(s, ga, be, ref, n):
        m = s / n
        v = var_of(ref, m, n)
        sc = lax.rsqrt(v + _EPS_BN) * ga
        return sc, be - m * sc

    def boundary(s, ga, be, hx_ref, x1o_col, Wn, bn_, more):
        sc, sh = bn_coefs(s, ga, be, agg_ref, N)

        def body(g, s2):
            hx = _rd(agg_ref, g) * sc + sh
            _wr(hx_ref, g, hx)
            x1o_ref[pl.ds(g, 1), pl.ds(x1o_col, 30)] = jnp.max(
                hx, axis=0, keepdims=True)
            if not more:
                return s2
            agg = gcn_agg(g, _mmh(hx, Wn)) + bn_
            _wr(agg_ref, g, agg)
            return s2 + jnp.sum(agg, axis=0, keepdims=True)
        return lax.fori_loop(0, G, body, z30)

    s = boundary(s, g11_ref[...], be11_ref[...], hx1_ref, 0,
                 W12_ref[...], b12_ref[...], True)
    s = boundary(s, g12_ref[...], be12_ref[...], hx2_ref, 30,
                 W13_ref[...], b13_ref[...], True)
    boundary(s, g13_ref[...], be13_ref[...], hx3_ref, 60,
             None, None, False)

    bel_ref[...] = b0_ref[...]
    betav = betav_ref[...]
    mseg = mseg_ref[...]

    def bp_body(g, c):
        a = _rd(A_ref, g)
        lb0 = _rd(logb0_ref, g)
        bel = _rd(bel_ref, g)
        h = _ctr(a, bel, 0, 0)
        e = jnp.exp(betav * h + lb0)
        _wr(bel_ref, g, e / _mmh(e, mseg))
        return c
    for _ in range(5):
        lax.fori_loop(0, G, bp_body, 0)

    Wp = Wp_ref[...]; bpb = bpb_ref[...]
    eyeK = (lax.broadcasted_iota(jnp.int32, (K, K), 0) ==
            lax.broadcasted_iota(jnp.int32, (K, K), 1)).astype(jnp.float32)

    def pool_body(g, carry):
        ml, ent = carry
        a = _rd(A_ref, g)
        S = _mmh(_rd(bel_ref, g), Wp, _DEF) + bpb
        t = _ctr(a, S, 1, 0, _DEF)
        ml = ml + jnp.sum(t * S)
        mx = jnp.max(S, axis=-1, keepdims=True)
        es = jnp.exp(S - mx)
        sm = es / jnp.sum(es, axis=-1, keepdims=True)
        ent = ent + jnp.sum(-sm * jnp.log(sm + 1e-15))
        _wr(p1x_ref, g, _ctr(sm, _rd(hx3_ref, g), 0, 0, _DEF))
        _wr(padj_ref, g, _ctr(sm, _ctr(a, sm, 1, 0, _DEF), 0, 0, _DEF))
        return ml, ent
    ml, ent = lax.fori_loop(0, G, pool_body,
                            (jnp.float32(0.0), jnp.float32(0.0)))
    p1_ml = -ml / E
    p1_el = ent / N

    ones_colK = jnp.ones((K, 1), jnp.float32)

    def dgcn_agg(g, z):
        a2 = _rd(padj_ref, g) + eyeK
        nr = lax.rsqrt(_ctr(a2, ones_colK, 1, 0) + 1e-9)
        return nr * _ctr(a2, nr * z, 1, 0, _DEF)

    def d1_body(g, s2):
        agg = dgcn_agg(g, _mmh(_rd(p1x_ref, g), W21_ref[...])) + b21_ref[...]
        _wr(agg2_ref, g, agg)
        return s2 + jnp.sum(agg, axis=0, keepdims=True)
    s = lax.fori_loop(0, G, d1_body, z30)

    NK = G * K

    def dboundary(s, ga, be, hx_ref, x2o_col, Wn, bn_, more):
        sc, sh = bn_coefs(s, ga, be, agg2_ref, NK)

        def body(g, s2):
            hx = _rd(agg2_ref, g) * sc + sh
            _wr(hx_ref, g, hx)
            x2o_ref[pl.ds(g, 1), pl.ds(x2o_col, 30)] = jnp.max(
                hx, axis=0, keepdims=True)
            if not more:
                return s2
            agg = dgcn_agg(g, _mmh(hx, Wn)) + bn_
            _wr(agg2_ref, g, agg)
            return s2 + jnp.sum(agg, axis=0, keepdims=True)
        return lax.fori_loop(0, G, body, z30)

    s = dboundary(s, g21_ref[...], be21_ref[...], h21_ref, 0,
                  W22_ref[...], b22_ref[...], True)
    s = dboundary(s, g22_ref[...], be22_ref[...], h22_ref, 30,
                  W23_ref[...], b23_ref[...], True)
    dboundary(s, g23_ref[...], be23_ref[...], h23_ref, 60,
              None, None, False)

    conv_out = jnp.concatenate([x1o_ref[...], x2o_ref[...]], axis=-1)
    hid = jnp.maximum(_mmh(conv_out, Wf1_ref[...], _DEF) + bf1_ref[...], 0.0)
    out_ref[...] = _mmh(hid, Wf2_ref[...], _DEF) + bf2_ref[...]
    reg_ref[...] = jnp.reshape(p1_el + p1_ml, (1, 1))


def kernel(x, edge_index, edge_attr, W11, b11, W12, b12, W13, b13,
           g11, be11, g12, be12, g13, be13, g21, be21, g22, be22, g23, be23,
           Wp, bpb, W21, b21, W22, b22, W23, b23, Wf1, bf1, Wf2, bf2):
    src = edge_index[0].astype(jnp.int32)
    dst = edge_index[1].astype(jnp.int32)
    ew = edge_attr

    gid = src // NPG
    A = jnp.zeros((G, NPG, NPG), jnp.float32).at[gid, src % NPG, dst % NPG].add(ew)

    idx = jnp.arange(N, dtype=jnp.float32)[:, None]
    b0s = []
    for q in _QS:
        freqs = jnp.arange(1, q + 1, dtype=jnp.float32)[None, :]
        z = jnp.sin(idx * freqs * 0.7331)
        z = z - jnp.max(z, axis=-1, keepdims=True)
        ez = jnp.exp(z)
        b0s.append(ez / jnp.sum(ez, axis=-1, keepdims=True))
    b0 = jnp.concatenate(b0s, axis=-1).reshape(G, NPG, _QTOT)
    logb0 = jnp.log(b0 + 1e-9)
    betav = jnp.concatenate([jnp.full((q,), b, jnp.float32)
                             for q, b in zip(_QS, _BETAS)])[None]
    run_id = jnp.concatenate([jnp.full((q,), i, jnp.int32)
                              for i, q in enumerate(_QS)])
    mseg = (run_id[:, None] == run_id[None, :]).astype(jnp.float32)

    f32 = jnp.float32
    scratch = [
        pltpu.VMEM((G, NPG, 1), f32),
        pltpu.VMEM((G, NPG, 30), f32),
        pltpu.VMEM((G, NPG, 30), f32),
        pltpu.VMEM((G, NPG, 30), f32),
        pltpu.VMEM((G, NPG, 30), f32),
        pltpu.VMEM((G, NPG, _QTOT), f32),
        pltpu.VMEM((G, K, 30), f32),
        pltpu.VMEM((G, K, K), f32),
        pltpu.VMEM((G, K, 30), f32),
        pltpu.VMEM((G, K, 30), f32),
        pltpu.VMEM((G, K, 30), f32),
        pltpu.VMEM((G, K, 30), f32),
        pltpu.VMEM((G, 90), f32),
        pltpu.VMEM((G, 90), f32),
    ]

    out, reg = pl.pallas_call(
        _mega,
        out_shape=(jax.ShapeDtypeStruct((G, 6), jnp.float32),
                   jax.ShapeDtypeStruct((1, 1), jnp.float32)),
        scratch_shapes=scratch,
        compiler_params=pltpu.CompilerParams(
            vmem_limit_bytes=100 * 1024 * 1024),
    )(A, x.reshape(G, NPG, 3), b0, logb0, betav, mseg,
      W11, b11, W12, b12, W13, b13,
      g11, be11, g12, be12, g13, be13,
      g21, be21, g22, be22, g23, be23,
      Wp, bpb, W21, b21, W22, b22, W23, b23, Wf1, bf1, Wf2, bf2)
    return (out, jnp.reshape(reg, ()))
